# baseline (speedup 1.0000x reference)
import functools

import jax
import jax.numpy as jnp
from jax import lax
from jax.experimental import pallas as pl
from jax.experimental.pallas import tpu as pltpu
from jax.experimental.pallas import tpu_sc as plsc

B, D_IN, D_H, D_Z, K = 4096, 512, 1024, 256, 8192
NUM_LVLS = 4
BT = 512
NBT = B // BT
KT = 2048
NKT = K // KT

_BF = jnp.bfloat16
_F32 = jnp.float32


def _mm(a, b):
    return lax.dot_general(a.astype(_BF), b.astype(_BF),
                           (((1,), (0,)), ((), ())),
                           preferred_element_type=_F32)


def _mm_nt(a, b):
    return lax.dot_general(a.astype(_BF), b.astype(_BF),
                           (((1,), (1,)), ((), ())),
                           preferred_element_type=_F32)


def _enc_body(x_ref, w1_ref, b1_ref, w2_ref, b2_ref, ze_ref):
    h = jnp.maximum(_mm(x_ref[...], w1_ref[...]) + b1_ref[...], 0.0)
    ze_ref[...] = _mm(h, w2_ref[...]) + b2_ref[...]


def _encoder(x, w1, b1, w2, b2):
    return pl.pallas_call(
        _enc_body,
        grid=(NBT,),
        in_specs=[
            pl.BlockSpec((BT, D_IN), lambda i: (i, 0)),
            pl.BlockSpec((D_IN, D_H), lambda i: (0, 0)),
            pl.BlockSpec((1, D_H), lambda i: (0, 0)),
            pl.BlockSpec((D_H, D_Z), lambda i: (0, 0)),
            pl.BlockSpec((1, D_Z), lambda i: (0, 0)),
        ],
        out_specs=pl.BlockSpec((BT, D_Z), lambda i: (i, 0)),
        out_shape=jax.ShapeDtypeStruct((B, D_Z), _F32),
    )(x, w1.astype(_F32), b1.reshape(1, D_H), w2, b2.reshape(1, D_Z))


def _prep_body(cb_ref, cbb_ref, cc_ref):
    cb = cb_ref[0]
    cbb_ref[0] = cb.astype(_BF) * jnp.asarray(2, _BF)
    cc_ref[0, 0] = jnp.sum(cb * cb, axis=1)


def _prep_codebooks(codebooks):
    return pl.pallas_call(
        _prep_body,
        grid=(NUM_LVLS, NKT),
        in_specs=[pl.BlockSpec((1, KT, D_Z), lambda l, k: (l, k, 0))],
        out_specs=[
            pl.BlockSpec((1, KT, D_Z), lambda l, k: (l, k, 0)),
            pl.BlockSpec((1, 1, KT), lambda l, k: (l, 0, k)),
        ],
        out_shape=[
            jax.ShapeDtypeStruct((NUM_LVLS, K, D_Z), _BF),
            jax.ShapeDtypeStruct((NUM_LVLS, 1, K), _F32),
        ],
    )(codebooks)


def _dist_body(first_level, r_ref, q_ref, zq_in_ref, cbb_ref, cc_ref,
               idx_ref, r_out_ref, zq_out_ref, rr_rows_ref,
               minv_ref, mini_ref, rr_ref):
    k = pl.program_id(1)

    @pl.when(k == 0)
    def _init():
        if first_level:
            r0 = r_ref[...]
        else:
            r0 = r_ref[...] - q_ref[...]
            r_out_ref[...] = r0
            if zq_in_ref is None:
                zq_out_ref[...] = q_ref[...]
            else:
                zq_out_ref[...] = zq_in_ref[...] + q_ref[...]
        rr = jnp.sum(r0 * r0, axis=1, keepdims=True)
        rr_ref[...] = rr
        if not first_level:
            rr_rows_ref[0, 0, :] = rr[:, 0]
        minv_ref[...] = jnp.full((BT, 128), jnp.inf, _F32)
        mini_ref[...] = jnp.zeros((BT, 128), jnp.int32)

    r = r_ref[...] if first_level else r_out_ref[...]
    rb = r.astype(_BF)
    cbb = cbb_ref[0, pl.ds(k * KT, KT), :]
    mm2 = lax.dot_general(rb, cbb, (((1,), (1,)), ((), ())),
                          preferred_element_type=_F32)
    d = (rr_ref[...] - mm2) + cc_ref[0, 0, pl.ds(k * KT, KT)][None, :]

    minv = minv_ref[...]
    mini = mini_ref[...]
    for g in range(KT // 128):
        dg = d[:, g * 128:(g + 1) * 128]
        better = dg < minv
        minv = jnp.where(better, dg, minv)
        mini = jnp.where(better, jnp.int32(k * (KT // 128) + g), mini)
    minv_ref[...] = minv
    mini_ref[...] = mini

    @pl.when(k == NKT - 1)
    def _fin():
        m = jnp.min(minv, axis=1, keepdims=True)
        lane = lax.broadcasted_iota(jnp.int32, (BT, 128), 1)
        cand = jnp.where(minv == m, mini * 128 + lane, jnp.int32(2 ** 30))
        idx_ref[0, 0, :] = jnp.min(cand, axis=1)


def _dist_level(r, q_prev, zq_in, cbb, cc, lvl, row_off=0, bc=None):
    bc = bc if bc is not None else r.shape[0]
    nbt = bc // BT
    first_level = q_prev is None
    body = functools.partial(_dist_body, first_level)

    rq_spec = pl.BlockSpec((BT, D_Z), lambda i, k: (i, 0))
    row_spec = pl.BlockSpec((1, 1, BT), lambda i, k: (i, 0, 0))

    in_specs = [pl.BlockSpec((BT, D_Z), lambda i, k: (row_off + i, 0))]
    args = [r]
    if first_level:
        body2 = lambda r_ref, cbb_ref, cc_ref, *rest: body(
            r_ref, None, None, cbb_ref, cc_ref, *rest)
    elif zq_in is None:
        in_specs.append(rq_spec)
        args.append(q_prev)
        body2 = lambda r_ref, q_ref, cbb_ref, cc_ref, *rest: body(
            r_ref, q_ref, None, cbb_ref, cc_ref, *rest)
    else:
        in_specs += [rq_spec, rq_spec]
        args += [q_prev, zq_in]
        body2 = lambda r_ref, q_ref, zq_ref, cbb_ref, cc_ref, *rest: body(
            r_ref, q_ref, zq_ref, cbb_ref, cc_ref, *rest)
    in_specs += [
        pl.BlockSpec((1, K, D_Z), lambda i, k: (lvl, 0, 0)),
        pl.BlockSpec((1, 1, K), lambda i, k: (lvl, 0, 0)),
    ]
    args += [cbb, cc]

    out = pl.pallas_call(
        body2,
        grid=(nbt, NKT),
        in_specs=in_specs,
        out_specs=[row_spec, rq_spec, rq_spec, row_spec],
        out_shape=[
            jax.ShapeDtypeStruct((nbt, 1, BT), jnp.int32),
            jax.ShapeDtypeStruct((bc, D_Z), _F32),
            jax.ShapeDtypeStruct((bc, D_Z), _F32),
            jax.ShapeDtypeStruct((nbt, 1, BT), _F32),
        ],
        scratch_shapes=[
            pltpu.VMEM((BT, 128), _F32),
            pltpu.VMEM((BT, 128), jnp.int32),
            pltpu.VMEM((BT, 1), _F32),
        ],
        compiler_params=pltpu.CompilerParams(
            dimension_semantics=("arbitrary", "arbitrary")),
    )(*args)
    idx_rows, r_new, zq_out, vq_rows = out
    return idx_rows.reshape(bc), r_new, zq_out, vq_rows


NW = 32


def _sc_gather(table, idx):
    bc = idx.shape[0]
    bpw = bc // NW
    mesh = plsc.VectorSubcoreMesh(core_axis_name="c", subcore_axis_name="s")

    @functools.partial(
        pl.kernel, mesh=mesh,
        out_type=jax.ShapeDtypeStruct((bc, D_Z), _F32),
        scratch_types=[
            pltpu.VMEM((bpw,), jnp.int32),
            pltpu.VMEM((bpw, D_Z), _F32),
            pltpu.SemaphoreType.DMA,
        ],
    )
    def k(table_hbm, idx_hbm, out_hbm, idx_v, rows_v, sem):
        wid = lax.axis_index("s") * 2 + lax.axis_index("c")
        base = wid * bpw
        pltpu.sync_copy(idx_hbm.at[pl.ds(base, bpw)], idx_v)
        pltpu.async_copy(table_hbm.at[idx_v], rows_v, sem).wait()
        pltpu.sync_copy(rows_v, out_hbm.at[pl.ds(base, bpw)])

    return k(table, idx)


def _dec_body(x_ref, ze_ref, zq_in_ref, q4_ref, r4_ref,
              w1_ref, b1_ref, w2_ref, b2_ref,
              xhat_ref, rec_rows_ref, vq_rows_ref):
    zq = zq_in_ref[...] + q4_ref[...]
    zq_st = ze_ref[...] + (zq - ze_ref[...])
    rfin = r4_ref[...] - q4_ref[...]
    vq_rows_ref[0, 0, :] = jnp.sum(rfin * rfin, axis=1)
    h2 = jnp.maximum(_mm(zq_st, w1_ref[...]) + b1_ref[...], 0.0)
    xh = _mm(h2, w2_ref[...]) + b2_ref[...]
    xhat_ref[...] = xh
    e = xh - x_ref[...]
    rec_rows_ref[0, 0, :] = jnp.sum(e * e, axis=1)


def _decoder(x, ze, zq_in, q4, r4, w1, b1, w2, b2, row_off, xhat_prev=None):
    bc = q4.shape[0]
    nbt = bc // BT
    rq_spec = pl.BlockSpec((BT, D_Z), lambda i: (i, 0))
    row_spec = pl.BlockSpec((1, 1, BT), lambda i: (i, 0, 0))
    in_specs = [
        pl.BlockSpec((BT, D_IN), lambda i: (row_off + i, 0)),
        pl.BlockSpec((BT, D_Z), lambda i: (row_off + i, 0)),
        rq_spec, rq_spec, rq_spec,
        pl.BlockSpec((D_Z, D_H), lambda i: (0, 0)),
        pl.BlockSpec((1, D_H), lambda i: (0, 0)),
        pl.BlockSpec((D_H, D_IN), lambda i: (0, 0)),
        pl.BlockSpec((1, D_IN), lambda i: (0, 0)),
    ]
    args = [x, ze, zq_in, q4, r4,
            w1, b1.reshape(1, D_H), w2, b2.reshape(1, D_IN)]
    aliases = {}
    if xhat_prev is not None:
        in_specs.append(pl.BlockSpec(memory_space=pl.ANY))
        args.append(xhat_prev)
        aliases = {9: 0}

    def body(*refs):
        if xhat_prev is not None:
            refs = refs[:9] + refs[10:]
        _dec_body(*refs)

    return pl.pallas_call(
        body,
        grid=(nbt,),
        in_specs=in_specs,
        out_specs=[
            pl.BlockSpec((BT, D_IN), lambda i: (row_off + i, 0)),
            row_spec, row_spec,
        ],
        out_shape=[
            jax.ShapeDtypeStruct((B, D_IN), _F32),
            jax.ShapeDtypeStruct((nbt, 1, BT), _F32),
            jax.ShapeDtypeStruct((nbt, 1, BT), _F32),
        ],
        input_output_aliases=aliases,
    )(*args)


NCHUNK = 4
CB = B // NCHUNK


def kernel(x, enc_w1, enc_b1, enc_w2, enc_b2, codebooks,
           dec_w1, dec_b1, dec_w2, dec_b2):
    ze = _encoder(x, enc_w1, enc_b1, enc_w2, enc_b2)
    cbb, cc = _prep_codebooks(codebooks)

    codes_c, rec_c, vq_c = [], [], []
    x_hat = None
    nbt_c = CB // BT
    for c in range(NCHUNK):
        off = c * nbt_c

        idx1, _, _, _ = _dist_level(ze, None, None, cbb, cc, 0,
                                    row_off=off, bc=CB)
        q1 = _sc_gather(codebooks[0], idx1)

        idx2, r2, zq2, vq1 = _dist_level(ze, q1, None, cbb, cc, 1,
                                         row_off=off, bc=CB)
        q2 = _sc_gather(codebooks[1], idx2)

        idx3, r3, zq3, vq2 = _dist_level(r2, q2, zq2, cbb, cc, 2)
        q3 = _sc_gather(codebooks[2], idx3)

        idx4, r4, zq4, vq3 = _dist_level(r3, q3, zq3, cbb, cc, 3)
        q4 = _sc_gather(codebooks[3], idx4)

        x_hat, rec_rows, vq4_rows = _decoder(
            x, ze, zq4, q4, r4, dec_w1, dec_b1, dec_w2, dec_b2,
            row_off=off, xhat_prev=x_hat)

        codes_c.append(jnp.stack([idx1, idx2, idx3, idx4], axis=1))
        rec_c.append(rec_rows)
        vq_c.append((vq1, vq2, vq3, vq4_rows))

    n = jnp.float32(B * D_Z)
    vq_loss = jnp.float32(0.0)
    for lvl in range(NUM_LVLS):
        cl = sum(jnp.sum(v[lvl]) for v in vq_c) / n
        vq_loss = vq_loss + cl + 0.25 * cl
    recon_loss = sum(jnp.sum(r) for r in rec_c) / jnp.float32(B * D_IN)
    loss = recon_loss + 0.25 * vq_loss
    codes = jnp.concatenate(codes_c, axis=0)
    return loss, recon_loss, vq_loss, codes, x_hat


# speedup vs baseline: 1.1674x; 1.1674x over previous
import functools

import jax
import jax.numpy as jnp
from jax import lax
from jax.experimental import pallas as pl
from jax.experimental.pallas import tpu as pltpu
from jax.experimental.pallas import tpu_sc as plsc

B, D_IN, D_H, D_Z, K = 4096, 512, 1024, 256, 8192
NUM_LVLS = 4
BT = 512
NBT = B // BT
KT = 2048
NKT = K // KT

_BF = jnp.bfloat16
_F32 = jnp.float32


def _mm(a, b):
    return lax.dot_general(a.astype(_BF), b.astype(_BF),
                           (((1,), (0,)), ((), ())),
                           preferred_element_type=_F32)


def _mm_nt(a, b):
    return lax.dot_general(a.astype(_BF), b.astype(_BF),
                           (((1,), (1,)), ((), ())),
                           preferred_element_type=_F32)


def _enc_body(x_ref, w1_ref, b1_ref, w2_ref, b2_ref, ze_ref):
    h = jnp.maximum(_mm(x_ref[...], w1_ref[...]) + b1_ref[...], 0.0)
    ze_ref[...] = _mm(h, w2_ref[...]) + b2_ref[...]


def _encoder(x, w1, b1, w2, b2):
    return pl.pallas_call(
        _enc_body,
        grid=(NBT,),
        in_specs=[
            pl.BlockSpec((BT, D_IN), lambda i: (i, 0)),
            pl.BlockSpec((D_IN, D_H), lambda i: (0, 0)),
            pl.BlockSpec((1, D_H), lambda i: (0, 0)),
            pl.BlockSpec((D_H, D_Z), lambda i: (0, 0)),
            pl.BlockSpec((1, D_Z), lambda i: (0, 0)),
        ],
        out_specs=pl.BlockSpec((BT, D_Z), lambda i: (i, 0)),
        out_shape=jax.ShapeDtypeStruct((B, D_Z), _F32),
    )(x, w1.astype(_F32), b1.reshape(1, D_H), w2, b2.reshape(1, D_Z))


def _prep_body(cb_ref, cbb_ref, cc_ref):
    cb = cb_ref[0]
    cbb_ref[0] = cb.astype(_BF) * jnp.asarray(2, _BF)
    cc_ref[0, 0] = jnp.sum(cb * cb, axis=1)


def _prep_codebooks(codebooks):
    return pl.pallas_call(
        _prep_body,
        grid=(NUM_LVLS, NKT),
        in_specs=[pl.BlockSpec((1, KT, D_Z), lambda l, k: (l, k, 0))],
        out_specs=[
            pl.BlockSpec((1, KT, D_Z), lambda l, k: (l, k, 0)),
            pl.BlockSpec((1, 1, KT), lambda l, k: (l, 0, k)),
        ],
        out_shape=[
            jax.ShapeDtypeStruct((NUM_LVLS, K, D_Z), _BF),
            jax.ShapeDtypeStruct((NUM_LVLS, 1, K), _F32),
        ],
    )(codebooks)


def _dist_body(first_level, r_ref, q_ref, zq_in_ref, cbb_ref, cc_ref,
               idx_ref, r_out_ref, zq_out_ref, rr_rows_ref,
               minv_ref, mini_ref, rr_ref):
    k = pl.program_id(1)

    @pl.when(k == 0)
    def _init():
        if first_level:
            r0 = r_ref[...]
        else:
            r0 = r_ref[...] - q_ref[...]
            r_out_ref[...] = r0
            if zq_in_ref is None:
                zq_out_ref[...] = q_ref[...]
            else:
                zq_out_ref[...] = zq_in_ref[...] + q_ref[...]
        rr = jnp.sum(r0 * r0, axis=1, keepdims=True)
        rr_ref[...] = rr
        if not first_level:
            rr_rows_ref[0, 0, :] = rr[:, 0]
        minv_ref[...] = jnp.full((BT, 128), jnp.inf, _F32)
        mini_ref[...] = jnp.zeros((BT, 128), jnp.int32)

    r = r_ref[...] if first_level else r_out_ref[...]
    rb = r.astype(_BF)
    cbb = cbb_ref[0, pl.ds(k * KT, KT), :]
    mm2 = lax.dot_general(rb, cbb, (((1,), (1,)), ((), ())),
                          preferred_element_type=_F32)
    d = (rr_ref[...] - mm2) + cc_ref[0, 0, pl.ds(k * KT, KT)][None, :]

    minv = minv_ref[...]
    mini = mini_ref[...]
    for g in range(KT // 128):
        dg = d[:, g * 128:(g + 1) * 128]
        better = dg < minv
        mini = jnp.where(better, jnp.int32(k * (KT // 128) + g), mini)
        minv = jnp.minimum(dg, minv)
    minv_ref[...] = minv
    mini_ref[...] = mini

    @pl.when(k == NKT - 1)
    def _fin():
        m = jnp.min(minv, axis=1, keepdims=True)
        lane = lax.broadcasted_iota(jnp.int32, (BT, 128), 1)
        cand = jnp.where(minv == m, mini * 128 + lane, jnp.int32(2 ** 30))
        idx_ref[0, 0, :] = jnp.min(cand, axis=1)


def _dist_level(r, q_prev, zq_in, cbb, cc, lvl, row_off=0, bc=None):
    bc = bc if bc is not None else r.shape[0]
    nbt = bc // BT
    first_level = q_prev is None
    body = functools.partial(_dist_body, first_level)

    rq_spec = pl.BlockSpec((BT, D_Z), lambda i, k: (i, 0))
    row_spec = pl.BlockSpec((1, 1, BT), lambda i, k: (i, 0, 0))

    in_specs = [pl.BlockSpec((BT, D_Z), lambda i, k: (row_off + i, 0))]
    args = [r]
    if first_level:
        body2 = lambda r_ref, cbb_ref, cc_ref, *rest: body(
            r_ref, None, None, cbb_ref, cc_ref, *rest)
    elif zq_in is None:
        in_specs.append(rq_spec)
        args.append(q_prev)
        body2 = lambda r_ref, q_ref, cbb_ref, cc_ref, *rest: body(
            r_ref, q_ref, None, cbb_ref, cc_ref, *rest)
    else:
        in_specs += [rq_spec, rq_spec]
        args += [q_prev, zq_in]
        body2 = lambda r_ref, q_ref, zq_ref, cbb_ref, cc_ref, *rest: body(
            r_ref, q_ref, zq_ref, cbb_ref, cc_ref, *rest)
    in_specs += [
        pl.BlockSpec((1, K, D_Z), lambda i, k: (lvl, 0, 0)),
        pl.BlockSpec((1, 1, K), lambda i, k: (lvl, 0, 0)),
    ]
    args += [cbb, cc]

    out = pl.pallas_call(
        body2,
        grid=(nbt, NKT),
        in_specs=in_specs,
        out_specs=[row_spec, rq_spec, rq_spec, row_spec],
        out_shape=[
            jax.ShapeDtypeStruct((nbt, 1, BT), jnp.int32),
            jax.ShapeDtypeStruct((bc, D_Z), _F32),
            jax.ShapeDtypeStruct((bc, D_Z), _F32),
            jax.ShapeDtypeStruct((nbt, 1, BT), _F32),
        ],
        scratch_shapes=[
            pltpu.VMEM((BT, 128), _F32),
            pltpu.VMEM((BT, 128), jnp.int32),
            pltpu.VMEM((BT, 1), _F32),
        ],
        compiler_params=pltpu.CompilerParams(
            dimension_semantics=("arbitrary", "arbitrary")),
    )(*args)
    idx_rows, r_new, zq_out, vq_rows = out
    return idx_rows.reshape(bc), r_new, zq_out, vq_rows


NW = 32


def _sc_gather(table, idx):
    bc = idx.shape[0]
    bpw = bc // NW
    mesh = plsc.VectorSubcoreMesh(core_axis_name="c", subcore_axis_name="s")

    @functools.partial(
        pl.kernel, mesh=mesh,
        out_type=jax.ShapeDtypeStruct((bc, D_Z), _F32),
        scratch_types=[
            pltpu.VMEM((bpw,), jnp.int32),
            pltpu.VMEM((bpw, D_Z), _F32),
            pltpu.SemaphoreType.DMA,
        ],
    )
    def k(table_hbm, idx_hbm, out_hbm, idx_v, rows_v, sem):
        wid = lax.axis_index("s") * 2 + lax.axis_index("c")
        base = wid * bpw
        pltpu.sync_copy(idx_hbm.at[pl.ds(base, bpw)], idx_v)
        pltpu.async_copy(table_hbm.at[idx_v], rows_v, sem).wait()
        pltpu.sync_copy(rows_v, out_hbm.at[pl.ds(base, bpw)])

    return k(table, idx)


def _dec_body(x_ref, ze_ref, zq_in_ref, q4_ref, r4_ref,
              w1_ref, b1_ref, w2_ref, b2_ref,
              xhat_ref, rec_rows_ref, vq_rows_ref):
    zq = zq_in_ref[...] + q4_ref[...]
    zq_st = ze_ref[...] + (zq - ze_ref[...])
    rfin = r4_ref[...] - q4_ref[...]
    vq_rows_ref[0, 0, :] = jnp.sum(rfin * rfin, axis=1)
    h2 = jnp.maximum(_mm(zq_st, w1_ref[...]) + b1_ref[...], 0.0)
    xh = _mm(h2, w2_ref[...]) + b2_ref[...]
    xhat_ref[...] = xh
    e = xh - x_ref[...]
    rec_rows_ref[0, 0, :] = jnp.sum(e * e, axis=1)


def _decoder(x, ze, zq_in, q4, r4, w1, b1, w2, b2, row_off, xhat_prev=None):
    bc = q4.shape[0]
    nbt = bc // BT
    rq_spec = pl.BlockSpec((BT, D_Z), lambda i: (i, 0))
    row_spec = pl.BlockSpec((1, 1, BT), lambda i: (i, 0, 0))
    in_specs = [
        pl.BlockSpec((BT, D_IN), lambda i: (row_off + i, 0)),
        pl.BlockSpec((BT, D_Z), lambda i: (row_off + i, 0)),
        rq_spec, rq_spec, rq_spec,
        pl.BlockSpec((D_Z, D_H), lambda i: (0, 0)),
        pl.BlockSpec((1, D_H), lambda i: (0, 0)),
        pl.BlockSpec((D_H, D_IN), lambda i: (0, 0)),
        pl.BlockSpec((1, D_IN), lambda i: (0, 0)),
    ]
    args = [x, ze, zq_in, q4, r4,
            w1, b1.reshape(1, D_H), w2, b2.reshape(1, D_IN)]
    aliases = {}
    if xhat_prev is not None:
        in_specs.append(pl.BlockSpec(memory_space=pl.ANY))
        args.append(xhat_prev)
        aliases = {9: 0}

    def body(*refs):
        if xhat_prev is not None:
            refs = refs[:9] + refs[10:]
        _dec_body(*refs)

    return pl.pallas_call(
        body,
        grid=(nbt,),
        in_specs=in_specs,
        out_specs=[
            pl.BlockSpec((BT, D_IN), lambda i: (row_off + i, 0)),
            row_spec, row_spec,
        ],
        out_shape=[
            jax.ShapeDtypeStruct((B, D_IN), _F32),
            jax.ShapeDtypeStruct((nbt, 1, BT), _F32),
            jax.ShapeDtypeStruct((nbt, 1, BT), _F32),
        ],
        input_output_aliases=aliases,
    )(*args)


NCHUNK = 2
CB = B // NCHUNK


def kernel(x, enc_w1, enc_b1, enc_w2, enc_b2, codebooks,
           dec_w1, dec_b1, dec_w2, dec_b2):
    ze = _encoder(x, enc_w1, enc_b1, enc_w2, enc_b2)
    cbb, cc = _prep_codebooks(codebooks)

    codes_c, rec_c, vq_c = [], [], []
    x_hat = None
    nbt_c = CB // BT
    for c in range(NCHUNK):
        off = c * nbt_c

        idx1, _, _, _ = _dist_level(ze, None, None, cbb, cc, 0,
                                    row_off=off, bc=CB)
        q1 = _sc_gather(codebooks[0], idx1)

        idx2, r2, zq2, vq1 = _dist_level(ze, q1, None, cbb, cc, 1,
                                         row_off=off, bc=CB)
        q2 = _sc_gather(codebooks[1], idx2)

        idx3, r3, zq3, vq2 = _dist_level(r2, q2, zq2, cbb, cc, 2)
        q3 = _sc_gather(codebooks[2], idx3)

        idx4, r4, zq4, vq3 = _dist_level(r3, q3, zq3, cbb, cc, 3)
        q4 = _sc_gather(codebooks[3], idx4)

        x_hat, rec_rows, vq4_rows = _decoder(
            x, ze, zq4, q4, r4, dec_w1, dec_b1, dec_w2, dec_b2,
            row_off=off, xhat_prev=x_hat)

        codes_c.append(jnp.stack([idx1, idx2, idx3, idx4], axis=1))
        rec_c.append(rec_rows)
        vq_c.append((vq1, vq2, vq3, vq4_rows))

    n = jnp.float32(B * D_Z)
    vq_loss = jnp.float32(0.0)
    for lvl in range(NUM_LVLS):
        cl = sum(jnp.sum(v[lvl]) for v in vq_c) / n
        vq_loss = vq_loss + cl + 0.25 * cl
    recon_loss = sum(jnp.sum(r) for r in rec_c) / jnp.float32(B * D_IN)
    loss = recon_loss + 0.25 * vq_loss
    codes = jnp.concatenate(codes_c, axis=0)
    return loss, recon_loss, vq_loss, codes, x_hat


# speedup vs baseline: 1.2724x; 1.0899x over previous
import functools

import jax
import jax.numpy as jnp
from jax import lax
from jax.experimental import pallas as pl
from jax.experimental.pallas import tpu as pltpu
from jax.experimental.pallas import tpu_sc as plsc

B, D_IN, D_H, D_Z, K = 4096, 512, 1024, 256, 8192
NUM_LVLS = 4
BT = 512
NBT = B // BT
KT = 4096
NKT = K // KT

_BF = jnp.bfloat16
_F32 = jnp.float32


def _mm(a, b):
    return lax.dot_general(a.astype(_BF), b.astype(_BF),
                           (((1,), (0,)), ((), ())),
                           preferred_element_type=_F32)


def _mm_nt(a, b):
    return lax.dot_general(a.astype(_BF), b.astype(_BF),
                           (((1,), (1,)), ((), ())),
                           preferred_element_type=_F32)


def _enc_body(x_ref, w1_ref, b1_ref, w2_ref, b2_ref, ze_ref):
    h = jnp.maximum(_mm(x_ref[...], w1_ref[...]) + b1_ref[...], 0.0)
    ze_ref[...] = _mm(h, w2_ref[...]) + b2_ref[...]


def _encoder(x, w1, b1, w2, b2):
    return pl.pallas_call(
        _enc_body,
        grid=(NBT,),
        in_specs=[
            pl.BlockSpec((BT, D_IN), lambda i: (i, 0)),
            pl.BlockSpec((D_IN, D_H), lambda i: (0, 0)),
            pl.BlockSpec((1, D_H), lambda i: (0, 0)),
            pl.BlockSpec((D_H, D_Z), lambda i: (0, 0)),
            pl.BlockSpec((1, D_Z), lambda i: (0, 0)),
        ],
        out_specs=pl.BlockSpec((BT, D_Z), lambda i: (i, 0)),
        out_shape=jax.ShapeDtypeStruct((B, D_Z), _F32),
    )(x, w1.astype(_F32), b1.reshape(1, D_H), w2, b2.reshape(1, D_Z))


def _prep_body(cb_ref, cbb_ref, cc_ref):
    cb = cb_ref[0]
    cbb_ref[0] = cb.astype(_BF) * jnp.asarray(2, _BF)
    cc_ref[0, 0] = jnp.sum(cb * cb, axis=1)


def _prep_codebooks(codebooks):
    return pl.pallas_call(
        _prep_body,
        grid=(NUM_LVLS, NKT),
        in_specs=[pl.BlockSpec((1, KT, D_Z), lambda l, k: (l, k, 0))],
        out_specs=[
            pl.BlockSpec((1, KT, D_Z), lambda l, k: (l, k, 0)),
            pl.BlockSpec((1, 1, KT), lambda l, k: (l, 0, k)),
        ],
        out_shape=[
            jax.ShapeDtypeStruct((NUM_LVLS, K, D_Z), _BF),
            jax.ShapeDtypeStruct((NUM_LVLS, 1, K), _F32),
        ],
    )(codebooks)


def _dist_body(first_level, r_ref, q_ref, zq_in_ref, cbb_ref, cc_ref,
               idx_ref, r_out_ref, zq_out_ref, rr_rows_ref,
               minv_ref, mini_ref, rr_ref):
    k = pl.program_id(1)

    @pl.when(k == 0)
    def _init():
        if first_level:
            r0 = r_ref[...]
        else:
            r0 = r_ref[...] - q_ref[...]
            r_out_ref[...] = r0
            if zq_in_ref is None:
                zq_out_ref[...] = q_ref[...]
            else:
                zq_out_ref[...] = zq_in_ref[...] + q_ref[...]
        rr = jnp.sum(r0 * r0, axis=1, keepdims=True)
        rr_ref[...] = rr
        if not first_level:
            rr_rows_ref[0, 0, :] = rr[:, 0]
        minv_ref[...] = jnp.full((BT, 128), jnp.inf, _F32)
        mini_ref[...] = jnp.zeros((BT, 128), jnp.int32)

    r = r_ref[...] if first_level else r_out_ref[...]
    rb = r.astype(_BF)
    cbb = cbb_ref[0, pl.ds(k * KT, KT), :]
    mm2 = lax.dot_general(rb, cbb, (((1,), (1,)), ((), ())),
                          preferred_element_type=_F32)
    d = (rr_ref[...] - mm2) + cc_ref[0, 0, pl.ds(k * KT, KT)][None, :]

    minv = minv_ref[...]
    mini = mini_ref[...]
    for g in range(KT // 128):
        dg = d[:, g * 128:(g + 1) * 128]
        better = dg < minv
        mini = jnp.where(better, jnp.int32(k * (KT // 128) + g), mini)
        minv = jnp.minimum(dg, minv)
    minv_ref[...] = minv
    mini_ref[...] = mini

    @pl.when(k == NKT - 1)
    def _fin():
        m = jnp.min(minv, axis=1, keepdims=True)
        lane = lax.broadcasted_iota(jnp.int32, (BT, 128), 1)
        cand = jnp.where(minv == m, mini * 128 + lane, jnp.int32(2 ** 30))
        idx_ref[0, 0, :] = jnp.min(cand, axis=1)


def _dist_level(r, q_prev, zq_in, cbb, cc, lvl, row_off=0, bc=None):
    bc = bc if bc is not None else r.shape[0]
    nbt = bc // BT
    first_level = q_prev is None
    body = functools.partial(_dist_body, first_level)

    rq_spec = pl.BlockSpec((BT, D_Z), lambda i, k: (i, 0))
    row_spec = pl.BlockSpec((1, 1, BT), lambda i, k: (i, 0, 0))

    in_specs = [pl.BlockSpec((BT, D_Z), lambda i, k: (row_off + i, 0))]
    args = [r]
    if first_level:
        body2 = lambda r_ref, cbb_ref, cc_ref, *rest: body(
            r_ref, None, None, cbb_ref, cc_ref, *rest)
    elif zq_in is None:
        in_specs.append(rq_spec)
        args.append(q_prev)
        body2 = lambda r_ref, q_ref, cbb_ref, cc_ref, *rest: body(
            r_ref, q_ref, None, cbb_ref, cc_ref, *rest)
    else:
        in_specs += [rq_spec, rq_spec]
        args += [q_prev, zq_in]
        body2 = lambda r_ref, q_ref, zq_ref, cbb_ref, cc_ref, *rest: body(
            r_ref, q_ref, zq_ref, cbb_ref, cc_ref, *rest)
    in_specs += [
        pl.BlockSpec((1, K, D_Z), lambda i, k: (lvl, 0, 0)),
        pl.BlockSpec((1, 1, K), lambda i, k: (lvl, 0, 0)),
    ]
    args += [cbb, cc]

    out = pl.pallas_call(
        body2,
        grid=(nbt, NKT),
        in_specs=in_specs,
        out_specs=[row_spec, rq_spec, rq_spec, row_spec],
        out_shape=[
            jax.ShapeDtypeStruct((nbt, 1, BT), jnp.int32),
            jax.ShapeDtypeStruct((bc, D_Z), _F32),
            jax.ShapeDtypeStruct((bc, D_Z), _F32),
            jax.ShapeDtypeStruct((nbt, 1, BT), _F32),
        ],
        scratch_shapes=[
            pltpu.VMEM((BT, 128), _F32),
            pltpu.VMEM((BT, 128), jnp.int32),
            pltpu.VMEM((BT, 1), _F32),
        ],
        compiler_params=pltpu.CompilerParams(
            dimension_semantics=("arbitrary", "arbitrary")),
    )(*args)
    idx_rows, r_new, zq_out, vq_rows = out
    return idx_rows.reshape(bc), r_new, zq_out, vq_rows


NW = 32


def _sc_gather(table, idx):
    bc = idx.shape[0]
    bpw = bc // NW
    mesh = plsc.VectorSubcoreMesh(core_axis_name="c", subcore_axis_name="s")

    @functools.partial(
        pl.kernel, mesh=mesh,
        out_type=jax.ShapeDtypeStruct((bc, D_Z), _F32),
        scratch_types=[
            pltpu.VMEM((bpw,), jnp.int32),
            pltpu.VMEM((bpw, D_Z), _F32),
            pltpu.SemaphoreType.DMA,
        ],
    )
    def k(table_hbm, idx_hbm, out_hbm, idx_v, rows_v, sem):
        wid = lax.axis_index("s") * 2 + lax.axis_index("c")
        base = wid * bpw
        pltpu.sync_copy(idx_hbm.at[pl.ds(base, bpw)], idx_v)
        pltpu.async_copy(table_hbm.at[idx_v], rows_v, sem).wait()
        pltpu.sync_copy(rows_v, out_hbm.at[pl.ds(base, bpw)])

    return k(table, idx)


def _dec_body(x_ref, ze_ref, zq_in_ref, q4_ref, r4_ref,
              w1_ref, b1_ref, w2_ref, b2_ref,
              xhat_ref, rec_rows_ref, vq_rows_ref):
    zq = zq_in_ref[...] + q4_ref[...]
    zq_st = ze_ref[...] + (zq - ze_ref[...])
    rfin = r4_ref[...] - q4_ref[...]
    vq_rows_ref[0, 0, :] = jnp.sum(rfin * rfin, axis=1)
    h2 = jnp.maximum(_mm(zq_st, w1_ref[...]) + b1_ref[...], 0.0)
    xh = _mm(h2, w2_ref[...]) + b2_ref[...]
    xhat_ref[...] = xh
    e = xh - x_ref[...]
    rec_rows_ref[0, 0, :] = jnp.sum(e * e, axis=1)


def _decoder(x, ze, zq_in, q4, r4, w1, b1, w2, b2, row_off, xhat_prev=None):
    bc = q4.shape[0]
    nbt = bc // BT
    rq_spec = pl.BlockSpec((BT, D_Z), lambda i: (i, 0))
    row_spec = pl.BlockSpec((1, 1, BT), lambda i: (i, 0, 0))
    in_specs = [
        pl.BlockSpec((BT, D_IN), lambda i: (row_off + i, 0)),
        pl.BlockSpec((BT, D_Z), lambda i: (row_off + i, 0)),
        rq_spec, rq_spec, rq_spec,
        pl.BlockSpec((D_Z, D_H), lambda i: (0, 0)),
        pl.BlockSpec((1, D_H), lambda i: (0, 0)),
        pl.BlockSpec((D_H, D_IN), lambda i: (0, 0)),
        pl.BlockSpec((1, D_IN), lambda i: (0, 0)),
    ]
    args = [x, ze, zq_in, q4, r4,
            w1, b1.reshape(1, D_H), w2, b2.reshape(1, D_IN)]
    aliases = {}
    if xhat_prev is not None:
        in_specs.append(pl.BlockSpec(memory_space=pl.ANY))
        args.append(xhat_prev)
        aliases = {9: 0}

    def body(*refs):
        if xhat_prev is not None:
            refs = refs[:9] + refs[10:]
        _dec_body(*refs)

    return pl.pallas_call(
        body,
        grid=(nbt,),
        in_specs=in_specs,
        out_specs=[
            pl.BlockSpec((BT, D_IN), lambda i: (row_off + i, 0)),
            row_spec, row_spec,
        ],
        out_shape=[
            jax.ShapeDtypeStruct((B, D_IN), _F32),
            jax.ShapeDtypeStruct((nbt, 1, BT), _F32),
            jax.ShapeDtypeStruct((nbt, 1, BT), _F32),
        ],
        input_output_aliases=aliases,
    )(*args)


NCHUNK = 2
CB = B // NCHUNK


def kernel(x, enc_w1, enc_b1, enc_w2, enc_b2, codebooks,
           dec_w1, dec_b1, dec_w2, dec_b2):
    ze = _encoder(x, enc_w1, enc_b1, enc_w2, enc_b2)
    cbb, cc = _prep_codebooks(codebooks)

    codes_c, rec_c, vq_c = [], [], []
    x_hat = None
    nbt_c = CB // BT
    for c in range(NCHUNK):
        off = c * nbt_c

        idx1, _, _, _ = _dist_level(ze, None, None, cbb, cc, 0,
                                    row_off=off, bc=CB)
        q1 = _sc_gather(codebooks[0], idx1)

        idx2, r2, zq2, vq1 = _dist_level(ze, q1, None, cbb, cc, 1,
                                         row_off=off, bc=CB)
        q2 = _sc_gather(codebooks[1], idx2)

        idx3, r3, zq3, vq2 = _dist_level(r2, q2, zq2, cbb, cc, 2)
        q3 = _sc_gather(codebooks[2], idx3)

        idx4, r4, zq4, vq3 = _dist_level(r3, q3, zq3, cbb, cc, 3)
        q4 = _sc_gather(codebooks[3], idx4)

        x_hat, rec_rows, vq4_rows = _decoder(
            x, ze, zq4, q4, r4, dec_w1, dec_b1, dec_w2, dec_b2,
            row_off=off, xhat_prev=x_hat)

        codes_c.append(jnp.stack([idx1, idx2, idx3, idx4], axis=1))
        rec_c.append(rec_rows)
        vq_c.append((vq1, vq2, vq3, vq4_rows))

    n = jnp.float32(B * D_Z)
    vq_loss = jnp.float32(0.0)
    for lvl in range(NUM_LVLS):
        cl = sum(jnp.sum(v[lvl]) for v in vq_c) / n
        vq_loss = vq_loss + cl + 0.25 * cl
    recon_loss = sum(jnp.sum(r) for r in rec_c) / jnp.float32(B * D_IN)
    loss = recon_loss + 0.25 * vq_loss
    codes = jnp.concatenate(codes_c, axis=0)
    return loss, recon_loss, vq_loss, codes, x_hat


# speedup vs baseline: 1.2745x; 1.0017x over previous
import functools

import jax
import jax.numpy as jnp
from jax import lax
from jax.experimental import pallas as pl
from jax.experimental.pallas import tpu as pltpu
from jax.experimental.pallas import tpu_sc as plsc

B, D_IN, D_H, D_Z, K = 4096, 512, 1024, 256, 8192
NUM_LVLS = 4
BT = 512
NBT = B // BT
KT = 8192
NKT = K // KT

_BF = jnp.bfloat16
_F32 = jnp.float32


def _mm(a, b):
    return lax.dot_general(a.astype(_BF), b.astype(_BF),
                           (((1,), (0,)), ((), ())),
                           preferred_element_type=_F32)


def _mm_nt(a, b):
    return lax.dot_general(a.astype(_BF), b.astype(_BF),
                           (((1,), (1,)), ((), ())),
                           preferred_element_type=_F32)


def _enc_body(x_ref, w1_ref, b1_ref, w2_ref, b2_ref, ze_ref):
    h = jnp.maximum(_mm(x_ref[...], w1_ref[...]) + b1_ref[...], 0.0)
    ze_ref[...] = _mm(h, w2_ref[...]) + b2_ref[...]


def _encoder(x, w1, b1, w2, b2):
    return pl.pallas_call(
        _enc_body,
        grid=(NBT,),
        in_specs=[
            pl.BlockSpec((BT, D_IN), lambda i: (i, 0)),
            pl.BlockSpec((D_IN, D_H), lambda i: (0, 0)),
            pl.BlockSpec((1, D_H), lambda i: (0, 0)),
            pl.BlockSpec((D_H, D_Z), lambda i: (0, 0)),
            pl.BlockSpec((1, D_Z), lambda i: (0, 0)),
        ],
        out_specs=pl.BlockSpec((BT, D_Z), lambda i: (i, 0)),
        out_shape=jax.ShapeDtypeStruct((B, D_Z), _F32),
    )(x, w1.astype(_F32), b1.reshape(1, D_H), w2, b2.reshape(1, D_Z))


def _prep_body(cb_ref, cbb_ref, cc_ref):
    cb = cb_ref[0]
    cbb_ref[0] = cb.astype(_BF) * jnp.asarray(2, _BF)
    cc_ref[0, 0] = jnp.sum(cb * cb, axis=1)


def _prep_codebooks(codebooks):
    return pl.pallas_call(
        _prep_body,
        grid=(NUM_LVLS, NKT),
        in_specs=[pl.BlockSpec((1, KT, D_Z), lambda l, k: (l, k, 0))],
        out_specs=[
            pl.BlockSpec((1, KT, D_Z), lambda l, k: (l, k, 0)),
            pl.BlockSpec((1, 1, KT), lambda l, k: (l, 0, k)),
        ],
        out_shape=[
            jax.ShapeDtypeStruct((NUM_LVLS, K, D_Z), _BF),
            jax.ShapeDtypeStruct((NUM_LVLS, 1, K), _F32),
        ],
    )(codebooks)


def _dist_body(first_level, r_ref, q_ref, zq_in_ref, cbb_ref, cc_ref,
               idx_ref, r_out_ref, zq_out_ref, rr_rows_ref,
               minv_ref, mini_ref, rr_ref):
    k = pl.program_id(1)

    @pl.when(k == 0)
    def _init():
        if first_level:
            r0 = r_ref[...]
        else:
            r0 = r_ref[...] - q_ref[...]
            r_out_ref[...] = r0
            if zq_in_ref is None:
                zq_out_ref[...] = q_ref[...]
            else:
                zq_out_ref[...] = zq_in_ref[...] + q_ref[...]
        rr = jnp.sum(r0 * r0, axis=1, keepdims=True)
        rr_ref[...] = rr
        if not first_level:
            rr_rows_ref[0, 0, :] = rr[:, 0]
        minv_ref[...] = jnp.full((BT, 128), jnp.inf, _F32)
        mini_ref[...] = jnp.zeros((BT, 128), jnp.int32)

    r = r_ref[...] if first_level else r_out_ref[...]
    rb = r.astype(_BF)
    cbb = cbb_ref[0, pl.ds(k * KT, KT), :]
    mm2 = lax.dot_general(rb, cbb, (((1,), (1,)), ((), ())),
                          preferred_element_type=_F32)
    d = (rr_ref[...] - mm2) + cc_ref[0, 0, pl.ds(k * KT, KT)][None, :]

    minv = minv_ref[...]
    mini = mini_ref[...]
    for g in range(KT // 128):
        dg = d[:, g * 128:(g + 1) * 128]
        better = dg < minv
        mini = jnp.where(better, jnp.int32(k * (KT // 128) + g), mini)
        minv = jnp.minimum(dg, minv)
    minv_ref[...] = minv
    mini_ref[...] = mini

    @pl.when(k == NKT - 1)
    def _fin():
        m = jnp.min(minv, axis=1, keepdims=True)
        lane = lax.broadcasted_iota(jnp.int32, (BT, 128), 1)
        cand = jnp.where(minv == m, mini * 128 + lane, jnp.int32(2 ** 30))
        idx_ref[0, 0, :] = jnp.min(cand, axis=1)


def _dist_level(r, q_prev, zq_in, cbb, cc, lvl, row_off=0, bc=None):
    bc = bc if bc is not None else r.shape[0]
    nbt = bc // BT
    first_level = q_prev is None
    body = functools.partial(_dist_body, first_level)

    rq_spec = pl.BlockSpec((BT, D_Z), lambda i, k: (i, 0))
    row_spec = pl.BlockSpec((1, 1, BT), lambda i, k: (i, 0, 0))

    in_specs = [pl.BlockSpec((BT, D_Z), lambda i, k: (row_off + i, 0))]
    args = [r]
    if first_level:
        body2 = lambda r_ref, cbb_ref, cc_ref, *rest: body(
            r_ref, None, None, cbb_ref, cc_ref, *rest)
    elif zq_in is None:
        in_specs.append(rq_spec)
        args.append(q_prev)
        body2 = lambda r_ref, q_ref, cbb_ref, cc_ref, *rest: body(
            r_ref, q_ref, None, cbb_ref, cc_ref, *rest)
    else:
        in_specs += [rq_spec, rq_spec]
        args += [q_prev, zq_in]
        body2 = lambda r_ref, q_ref, zq_ref, cbb_ref, cc_ref, *rest: body(
            r_ref, q_ref, zq_ref, cbb_ref, cc_ref, *rest)
    in_specs += [
        pl.BlockSpec((1, K, D_Z), lambda i, k: (lvl, 0, 0)),
        pl.BlockSpec((1, 1, K), lambda i, k: (lvl, 0, 0)),
    ]
    args += [cbb, cc]

    out = pl.pallas_call(
        body2,
        grid=(nbt, NKT),
        in_specs=in_specs,
        out_specs=[row_spec, rq_spec, rq_spec, row_spec],
        out_shape=[
            jax.ShapeDtypeStruct((nbt, 1, BT), jnp.int32),
            jax.ShapeDtypeStruct((bc, D_Z), _F32),
            jax.ShapeDtypeStruct((bc, D_Z), _F32),
            jax.ShapeDtypeStruct((nbt, 1, BT), _F32),
        ],
        scratch_shapes=[
            pltpu.VMEM((BT, 128), _F32),
            pltpu.VMEM((BT, 128), jnp.int32),
            pltpu.VMEM((BT, 1), _F32),
        ],
        compiler_params=pltpu.CompilerParams(
            dimension_semantics=("arbitrary", "arbitrary")),
    )(*args)
    idx_rows, r_new, zq_out, vq_rows = out
    return idx_rows.reshape(bc), r_new, zq_out, vq_rows


NW = 32


def _sc_gather(table, idx):
    bc = idx.shape[0]
    bpw = bc // NW
    mesh = plsc.VectorSubcoreMesh(core_axis_name="c", subcore_axis_name="s")

    @functools.partial(
        pl.kernel, mesh=mesh,
        out_type=jax.ShapeDtypeStruct((bc, D_Z), _F32),
        scratch_types=[
            pltpu.VMEM((bpw,), jnp.int32),
            pltpu.VMEM((bpw, D_Z), _F32),
            pltpu.SemaphoreType.DMA,
        ],
    )
    def k(table_hbm, idx_hbm, out_hbm, idx_v, rows_v, sem):
        wid = lax.axis_index("s") * 2 + lax.axis_index("c")
        base = wid * bpw
        pltpu.sync_copy(idx_hbm.at[pl.ds(base, bpw)], idx_v)
        pltpu.async_copy(table_hbm.at[idx_v], rows_v, sem).wait()
        pltpu.sync_copy(rows_v, out_hbm.at[pl.ds(base, bpw)])

    return k(table, idx)


def _dec_body(x_ref, ze_ref, zq_in_ref, q4_ref, r4_ref,
              w1_ref, b1_ref, w2_ref, b2_ref,
              xhat_ref, rec_rows_ref, vq_rows_ref):
    zq = zq_in_ref[...] + q4_ref[...]
    zq_st = ze_ref[...] + (zq - ze_ref[...])
    rfin = r4_ref[...] - q4_ref[...]
    vq_rows_ref[0, 0, :] = jnp.sum(rfin * rfin, axis=1)
    h2 = jnp.maximum(_mm(zq_st, w1_ref[...]) + b1_ref[...], 0.0)
    xh = _mm(h2, w2_ref[...]) + b2_ref[...]
    xhat_ref[...] = xh
    e = xh - x_ref[...]
    rec_rows_ref[0, 0, :] = jnp.sum(e * e, axis=1)


def _decoder(x, ze, zq_in, q4, r4, w1, b1, w2, b2, row_off, xhat_prev=None):
    bc = q4.shape[0]
    nbt = bc // BT
    rq_spec = pl.BlockSpec((BT, D_Z), lambda i: (i, 0))
    row_spec = pl.BlockSpec((1, 1, BT), lambda i: (i, 0, 0))
    in_specs = [
        pl.BlockSpec((BT, D_IN), lambda i: (row_off + i, 0)),
        pl.BlockSpec((BT, D_Z), lambda i: (row_off + i, 0)),
        rq_spec, rq_spec, rq_spec,
        pl.BlockSpec((D_Z, D_H), lambda i: (0, 0)),
        pl.BlockSpec((1, D_H), lambda i: (0, 0)),
        pl.BlockSpec((D_H, D_IN), lambda i: (0, 0)),
        pl.BlockSpec((1, D_IN), lambda i: (0, 0)),
    ]
    args = [x, ze, zq_in, q4, r4,
            w1, b1.reshape(1, D_H), w2, b2.reshape(1, D_IN)]
    aliases = {}
    if xhat_prev is not None:
        in_specs.append(pl.BlockSpec(memory_space=pl.ANY))
        args.append(xhat_prev)
        aliases = {9: 0}

    def body(*refs):
        if xhat_prev is not None:
            refs = refs[:9] + refs[10:]
        _dec_body(*refs)

    return pl.pallas_call(
        body,
        grid=(nbt,),
        in_specs=in_specs,
        out_specs=[
            pl.BlockSpec((BT, D_IN), lambda i: (row_off + i, 0)),
            row_spec, row_spec,
        ],
        out_shape=[
            jax.ShapeDtypeStruct((B, D_IN), _F32),
            jax.ShapeDtypeStruct((nbt, 1, BT), _F32),
            jax.ShapeDtypeStruct((nbt, 1, BT), _F32),
        ],
        input_output_aliases=aliases,
    )(*args)


NCHUNK = 2
CB = B // NCHUNK


def kernel(x, enc_w1, enc_b1, enc_w2, enc_b2, codebooks,
           dec_w1, dec_b1, dec_w2, dec_b2):
    ze = _encoder(x, enc_w1, enc_b1, enc_w2, enc_b2)
    cbb, cc = _prep_codebooks(codebooks)

    codes_c, rec_c, vq_c = [], [], []
    x_hat = None
    nbt_c = CB // BT
    for c in range(NCHUNK):
        off = c * nbt_c

        idx1, _, _, _ = _dist_level(ze, None, None, cbb, cc, 0,
                                    row_off=off, bc=CB)
        q1 = _sc_gather(codebooks[0], idx1)

        idx2, r2, zq2, vq1 = _dist_level(ze, q1, None, cbb, cc, 1,
                                         row_off=off, bc=CB)
        q2 = _sc_gather(codebooks[1], idx2)

        idx3, r3, zq3, vq2 = _dist_level(r2, q2, zq2, cbb, cc, 2)
        q3 = _sc_gather(codebooks[2], idx3)

        idx4, r4, zq4, vq3 = _dist_level(r3, q3, zq3, cbb, cc, 3)
        q4 = _sc_gather(codebooks[3], idx4)

        x_hat, rec_rows, vq4_rows = _decoder(
            x, ze, zq4, q4, r4, dec_w1, dec_b1, dec_w2, dec_b2,
            row_off=off, xhat_prev=x_hat)

        codes_c.append(jnp.stack([idx1, idx2, idx3, idx4], axis=1))
        rec_c.append(rec_rows)
        vq_c.append((vq1, vq2, vq3, vq4_rows))

    n = jnp.float32(B * D_Z)
    vq_loss = jnp.float32(0.0)
    for lvl in range(NUM_LVLS):
        cl = sum(jnp.sum(v[lvl]) for v in vq_c) / n
        vq_loss = vq_loss + cl + 0.25 * cl
    recon_loss = sum(jnp.sum(r) for r in rec_c) / jnp.float32(B * D_IN)
    loss = recon_loss + 0.25 * vq_loss
    codes = jnp.concatenate(codes_c, axis=0)
    return loss, recon_loss, vq_loss, codes, x_hat
